# baseline (speedup 1.0000x reference)
import functools

import jax
import jax.numpy as jnp
from jax import lax
from jax.experimental import pallas as pl
from jax.experimental.pallas import tpu as pltpu
from jax.experimental.pallas import tpu_sc as plsc

EMBED = 64
SCALE = 8.0
NW = 32
NB = 4
LANES = 16
BLK = 128


@functools.lru_cache(maxsize=None)
def _build(n_rows: int, n_cols: int):
    assert n_rows == NW * BLK
    mesh = plsc.VectorSubcoreMesh(core_axis_name="c", subcore_axis_name="s")

    @functools.partial(
        pl.kernel,
        mesh=mesh,
        compiler_params=pltpu.CompilerParams(
            use_tc_tiling_on_sc=False, needs_layout_passes=False),
        out_type=jax.ShapeDtypeStruct(
            (n_cols, EMBED // 8, n_rows // 128, 8, 128), jnp.float32),
        scratch_types=(
            [pltpu.VMEM((n_cols, BLK), jnp.int32)]
            + [pltpu.VMEM((BLK, EMBED), jnp.float32) for _ in range(NB)]
            + [pltpu.VMEM((EMBED // 8, 8, 128), jnp.float32) for _ in range(NB)]
            + [pltpu.SemaphoreType.DMA for _ in range(2 * NB)]
        ),
    )
    def emb_kernel(table_hbm, idxt_hbm, out_hbm, idx_v, *rest):
        rows_in = rest[0:NB]
        slabs = rest[NB:2 * NB]
        gsem = rest[2 * NB:3 * NB]
        osem = rest[3 * NB:4 * NB]

        wid = lax.axis_index("s") * 2 + lax.axis_index("c")

        pltpu.sync_copy(idxt_hbm.at[:, pl.ds(wid * BLK, BLK)], idx_v)

        for b in range(NB):
            pltpu.async_copy(table_hbm.at[idx_v.at[b]], rows_in[b], gsem[b])

        def outer(g, carry):
            for b in range(NB):
                j = g * NB + b
                pltpu.make_async_copy(
                    table_hbm.at[idx_v.at[j]], rows_in[b], gsem[b]).wait()

                @pl.when(j >= NB)
                def _():
                    pltpu.make_async_copy(
                        slabs[b], out_hbm.at[j - NB, :, wid], osem[b]).wait()

                @plsc.parallel_loop(0, BLK, unroll=2)
                def token(i):
                    iota = lax.iota(jnp.int32, LANES)
                    ks_pat = iota % 8
                    il = jnp.full((LANES,), i, jnp.int32)
                    for c in range(EMBED // LANES):
                        kt_pat = iota // 8 + 2 * c
                        v = rows_in[b][i, pl.ds(c * LANES, LANES)] * SCALE
                        plsc.store_scatter(
                            slabs[b], [kt_pat, ks_pat, il], v)

                @pl.when(j + NB < n_cols)
                def _():
                    pltpu.async_copy(
                        table_hbm.at[idx_v.at[j + NB]], rows_in[b], gsem[b])

                pltpu.async_copy(slabs[b], out_hbm.at[j, :, wid], osem[b])
            return carry

        lax.fori_loop(0, n_cols // NB, outer, 0)

        for b in range(NB):
            pltpu.make_async_copy(
                slabs[b], out_hbm.at[n_cols - NB + b, :, wid], osem[b]).wait()

    return emb_kernel


@jax.jit
def kernel(x, table):
    n_rows, n_cols = x.shape
    xt = x.astype(jnp.int32).T
    out_lin = _build(n_rows, n_cols)(table, xt)
    out = out_lin.transpose(2, 4, 0, 1, 3).reshape(n_rows, n_cols, EMBED)
    return out


# speedup vs baseline: 1.0006x; 1.0006x over previous
import functools

import jax
import jax.numpy as jnp
from jax import lax
from jax.experimental import pallas as pl
from jax.experimental.pallas import tpu as pltpu
from jax.experimental.pallas import tpu_sc as plsc

EMBED = 64
SCALE = 8.0
NW = 32
NB = 4
LANES = 16
BLK = 128


@functools.lru_cache(maxsize=None)
def _build(n_rows: int, n_cols: int):
    assert n_rows == NW * BLK
    mesh = plsc.VectorSubcoreMesh(core_axis_name="c", subcore_axis_name="s")

    @functools.partial(
        pl.kernel,
        mesh=mesh,
        compiler_params=pltpu.CompilerParams(
            use_tc_tiling_on_sc=False, needs_layout_passes=False),
        out_type=jax.ShapeDtypeStruct(
            (n_cols, EMBED // 8, n_rows // 128, 8 * 128), jnp.float32),
        scratch_types=(
            [pltpu.VMEM((n_cols, BLK), jnp.int32)]
            + [pltpu.VMEM((BLK, EMBED), jnp.float32) for _ in range(NB)]
            + [pltpu.VMEM((EMBED // 8, 8 * 128), jnp.float32) for _ in range(NB)]
            + [pltpu.SemaphoreType.DMA for _ in range(2 * NB)]
        ),
    )
    def emb_kernel(table_hbm, idxt_hbm, out_hbm, idx_v, *rest):
        rows_in = rest[0:NB]
        slabs = rest[NB:2 * NB]
        gsem = rest[2 * NB:3 * NB]
        osem = rest[3 * NB:4 * NB]

        wid = lax.axis_index("s") * 2 + lax.axis_index("c")

        pltpu.sync_copy(idxt_hbm.at[:, pl.ds(wid * BLK, BLK)], idx_v)

        iota = lax.iota(jnp.int32, LANES)
        kt_pats = [iota // 8 + 2 * c for c in range(EMBED // LANES)]
        ks_base = (iota % 8) * BLK

        for b in range(NB):
            pltpu.async_copy(table_hbm.at[idx_v.at[b]], rows_in[b], gsem[b])

        def outer(g, carry):
            for b in range(NB):
                j = g * NB + b
                pltpu.make_async_copy(
                    table_hbm.at[idx_v.at[j]], rows_in[b], gsem[b]).wait()

                @pl.when(j >= NB)
                def _():
                    pltpu.make_async_copy(
                        slabs[b], out_hbm.at[j - NB, :, wid], osem[b]).wait()

                @plsc.parallel_loop(0, BLK, unroll=8)
                def token(i):
                    pos = ks_base + i
                    for c in range(EMBED // LANES):
                        v = rows_in[b][i, pl.ds(c * LANES, LANES)] * SCALE
                        plsc.store_scatter(slabs[b], [kt_pats[c], pos], v)

                @pl.when(j + NB < n_cols)
                def _():
                    pltpu.async_copy(
                        table_hbm.at[idx_v.at[j + NB]], rows_in[b], gsem[b])

                pltpu.async_copy(slabs[b], out_hbm.at[j, :, wid], osem[b])
            return carry

        lax.fori_loop(0, n_cols // NB, outer, 0)

        for b in range(NB):
            pltpu.make_async_copy(
                slabs[b], out_hbm.at[n_cols - NB + b, :, wid], osem[b]).wait()

    return emb_kernel


@jax.jit
def kernel(x, table):
    n_rows, n_cols = x.shape
    xt = x.astype(jnp.int32).T
    out_lin = _build(n_rows, n_cols)(table, xt)
    out = (out_lin.reshape(n_cols, EMBED // 8, n_rows // 128, 8, 128)
           .transpose(2, 4, 0, 1, 3).reshape(n_rows, n_cols, EMBED))
    return out


# speedup vs baseline: 1.7386x; 1.7377x over previous
import functools

import jax
import jax.numpy as jnp
from jax import lax
from jax.experimental import pallas as pl
from jax.experimental.pallas import tpu as pltpu
from jax.experimental.pallas import tpu_sc as plsc

EMBED = 64
SCALE = 8.0
NW = 32
NB = 4
LANES = 16
BLK = 128


@functools.lru_cache(maxsize=None)
def _build(n_rows: int, n_cols: int):
    assert n_rows == NW * BLK
    mesh = plsc.VectorSubcoreMesh(core_axis_name="c", subcore_axis_name="s")

    @functools.partial(
        pl.kernel,
        mesh=mesh,
        compiler_params=pltpu.CompilerParams(
            use_tc_tiling_on_sc=False, needs_layout_passes=False),
        out_type=jax.ShapeDtypeStruct(
            (n_cols, EMBED // 8, n_rows // 128, 8, 128), jnp.float32),
        scratch_types=(
            [pltpu.VMEM((n_cols, BLK), jnp.int32)]
            + [pltpu.VMEM((BLK, EMBED), jnp.float32) for _ in range(NB)]
            + [pltpu.VMEM((EMBED // 8, 8, 129), jnp.float32) for _ in range(NB)]
            + [pltpu.SemaphoreType.DMA for _ in range(2 * NB)]
        ),
    )
    def emb_kernel(table_hbm, idxt_hbm, out_hbm, idx_v, *rest):
        rows_in = rest[0:NB]
        slabs = rest[NB:2 * NB]
        gsem = rest[2 * NB:3 * NB]
        osem = rest[3 * NB:4 * NB]

        wid = lax.axis_index("s") * 2 + lax.axis_index("c")

        pltpu.sync_copy(idxt_hbm.at[:, pl.ds(wid * BLK, BLK)], idx_v)

        iota = lax.iota(jnp.int32, LANES)
        kt_pats = [iota // 8 + 2 * c for c in range(EMBED // LANES)]
        ks_pat = iota % 8

        for b in range(NB):
            pltpu.async_copy(table_hbm.at[idx_v.at[b]], rows_in[b], gsem[b])

        def outer(g, carry):
            for b in range(NB):
                j = g * NB + b
                pltpu.make_async_copy(
                    table_hbm.at[idx_v.at[j]], rows_in[b], gsem[b]).wait()

                @pl.when(j >= NB)
                def _():
                    pltpu.make_async_copy(
                        slabs[b].at[:, :, pl.ds(0, 128)],
                        out_hbm.at[j - NB, :, wid], osem[b]).wait()

                @plsc.parallel_loop(0, BLK, unroll=4)
                def token(i):
                    il = jnp.full((LANES,), i, jnp.int32)
                    for c in range(EMBED // LANES):
                        v = rows_in[b][i, pl.ds(c * LANES, LANES)] * SCALE
                        plsc.store_scatter(
                            slabs[b], [kt_pats[c], ks_pat, il], v)

                @pl.when(j + NB < n_cols)
                def _():
                    pltpu.async_copy(
                        table_hbm.at[idx_v.at[j + NB]], rows_in[b], gsem[b])

                pltpu.async_copy(
                    slabs[b].at[:, :, pl.ds(0, 128)],
                    out_hbm.at[j, :, wid], osem[b])
            return carry

        lax.fori_loop(0, n_cols // NB, outer, 0)

        for b in range(NB):
            pltpu.make_async_copy(
                slabs[b].at[:, :, pl.ds(0, 128)],
                out_hbm.at[n_cols - NB + b, :, wid], osem[b]).wait()

    return emb_kernel


@jax.jit
def kernel(x, table):
    n_rows, n_cols = x.shape
    xt = x.astype(jnp.int32).T
    out_lin = _build(n_rows, n_cols)(table, xt)
    out = out_lin.transpose(2, 4, 0, 1, 3).reshape(n_rows, n_cols, EMBED)
    return out


# speedup vs baseline: 1.7438x; 1.0030x over previous
import functools

import jax
import jax.numpy as jnp
from jax import lax
from jax.experimental import pallas as pl
from jax.experimental.pallas import tpu as pltpu
from jax.experimental.pallas import tpu_sc as plsc
from jax.experimental import layout as jax_layout

EMBED = 64
SCALE = 8.0
NW = 32
NB = 4
LANES = 16
BLK = 128


@functools.lru_cache(maxsize=None)
def _build(n_rows: int, n_cols: int):
    assert n_rows == NW * BLK
    mesh = plsc.VectorSubcoreMesh(core_axis_name="c", subcore_axis_name="s")

    @functools.partial(
        pl.kernel,
        mesh=mesh,
        compiler_params=pltpu.CompilerParams(
            use_tc_tiling_on_sc=False, needs_layout_passes=False),
        out_type=jax.ShapeDtypeStruct(
            (n_cols, EMBED // 8, n_rows // 128, 8, 128), jnp.float32),
        scratch_types=(
            [pltpu.VMEM((n_cols, BLK), jnp.int32)]
            + [pltpu.VMEM((BLK, EMBED), jnp.float32) for _ in range(NB)]
            + [pltpu.VMEM((EMBED // 8, 8, 129), jnp.float32) for _ in range(NB)]
            + [pltpu.SemaphoreType.DMA for _ in range(2 * NB)]
        ),
    )
    def emb_kernel(table_hbm, idxt_hbm, out_hbm, idx_v, *rest):
        rows_in = rest[0:NB]
        slabs = rest[NB:2 * NB]
        gsem = rest[2 * NB:3 * NB]
        osem = rest[3 * NB:4 * NB]

        wid = lax.axis_index("s") * 2 + lax.axis_index("c")

        pltpu.sync_copy(idxt_hbm.at[:, pl.ds(wid * BLK, BLK)], idx_v)

        iota = lax.iota(jnp.int32, LANES)
        kt_pats = [iota // 8 + 2 * c for c in range(EMBED // LANES)]
        ks_pat = iota % 8

        for b in range(NB):
            pltpu.async_copy(table_hbm.at[idx_v.at[b]], rows_in[b], gsem[b])

        def outer(g, carry):
            for b in range(NB):
                j = g * NB + b
                pltpu.make_async_copy(
                    table_hbm.at[idx_v.at[j]], rows_in[b], gsem[b]).wait()

                @pl.when(j >= NB)
                def _():
                    pltpu.make_async_copy(
                        slabs[b].at[:, :, pl.ds(0, 128)],
                        out_hbm.at[j - NB, :, wid], osem[b]).wait()

                @plsc.parallel_loop(0, BLK, unroll=4)
                def token(i):
                    il = jnp.full((LANES,), i, jnp.int32)
                    for c in range(EMBED // LANES):
                        v = rows_in[b][i, pl.ds(c * LANES, LANES)] * SCALE
                        plsc.store_scatter(
                            slabs[b], [kt_pats[c], ks_pat, il], v)

                @pl.when(j + NB < n_cols)
                def _():
                    pltpu.async_copy(
                        table_hbm.at[idx_v.at[j + NB]], rows_in[b], gsem[b])

                pltpu.async_copy(
                    slabs[b].at[:, :, pl.ds(0, 128)],
                    out_hbm.at[j, :, wid], osem[b])
            return carry

        lax.fori_loop(0, n_cols // NB, outer, 0)

        for b in range(NB):
            pltpu.make_async_copy(
                slabs[b].at[:, :, pl.ds(0, 128)],
                out_hbm.at[n_cols - NB + b, :, wid], osem[b]).wait()

    return emb_kernel


@jax.jit
def kernel(x, table):
    n_rows, n_cols = x.shape
    xt = x.astype(jnp.int32).T
    tbl = jax_layout.with_layout_constraint(
        table, jax_layout.Layout((1, 0), ((8, EMBED),)))
    out_lin = _build(n_rows, n_cols)(tbl, xt)
    out = out_lin.transpose(2, 4, 0, 1, 3).reshape(n_rows, n_cols, EMBED)
    return out
